# baseline (speedup 1.0000x reference)
import functools

import jax
import jax.numpy as jnp
from jax import lax
from jax.experimental import pallas as pl
from jax.experimental.pallas import tpu as pltpu
from jax.experimental.pallas import tpu_sc as plsc

HID = 128
GAMMA = 12.0
ERANGE = (12.0 + 2.0) / HID
PI = 3.141592653589793
PHASE_SCALE = PI / ERANGE

NC = 2
NS = 16
NW = NC * NS
LANES = 16


def _trig_body(rel_ref, cos_ref, sin_ref):
    phase = rel_ref[...] * jnp.float32(PHASE_SCALE)
    cos_ref[...] = jnp.cos(phase)
    sin_ref[...] = jnp.sin(phase)


def _trig_tables(rel_emb):
    n, h = rel_emb.shape
    out = jax.ShapeDtypeStruct((n, h), jnp.float32)
    return pl.pallas_call(_trig_body, out_shape=(out, out))(rel_emb)


def _sc_body(bpw, ent_hbm, cos_hbm, sin_hbm, idxh_hbm, idxr_hbm, idxt_hbm,
             out_hbm, idxh_v, idxr_v, idxt_v, h_v, t_v, c_v, s_v, out_v,
             sem0, sem1, sem2, sem3):
    wid = lax.axis_index("s") * NC + lax.axis_index("c")
    base = wid * bpw
    pltpu.sync_copy(idxh_hbm.at[pl.ds(base, bpw)], idxh_v)
    pltpu.sync_copy(idxr_hbm.at[pl.ds(base, bpw)], idxr_v)
    pltpu.sync_copy(idxt_hbm.at[pl.ds(base, bpw)], idxt_v)
    ch = pltpu.async_copy(ent_hbm.at[idxh_v], h_v, sem0)
    ct = pltpu.async_copy(ent_hbm.at[idxt_v], t_v, sem1)
    cc = pltpu.async_copy(cos_hbm.at[idxr_v], c_v, sem2)
    cs = pltpu.async_copy(sin_hbm.at[idxr_v], s_v, sem3)
    ch.wait()
    ct.wait()
    cc.wait()
    cs.wait()

    half = jnp.int32(HID)
    for g in range(bpw // LANES):
        row = lax.broadcasted_iota(jnp.int32, (LANES,), 0) + jnp.int32(g * LANES)

        def dim_step(d, acc, row=row):
            cd = jnp.full((LANES,), d, jnp.int32)
            cd2 = cd + half
            rh = plsc.load_gather(h_v, [row, cd])
            ih = plsc.load_gather(h_v, [row, cd2])
            rt = plsc.load_gather(t_v, [row, cd])
            it = plsc.load_gather(t_v, [row, cd2])
            c = plsc.load_gather(c_v, [row, cd])
            s = plsc.load_gather(s_v, [row, cd])
            re = rh * c - ih * s - rt
            im = rh * s + ih * c - it
            x = re * re + im * im + jnp.float32(1e-30)
            yi = jnp.int32(0x5F3759DF) - lax.shift_right_logical(
                plsc.bitcast(x, jnp.int32), jnp.int32(1))
            y = plsc.bitcast(yi, jnp.float32)
            hx = jnp.float32(0.5) * x
            y = y * (jnp.float32(1.5) - hx * y * y)
            y = y * (jnp.float32(1.5) - hx * y * y)
            y = y * (jnp.float32(1.5) - hx * y * y)
            return acc + x * y

        acc = lax.fori_loop(0, HID, dim_step, jnp.zeros((LANES,), jnp.float32))
        out_v[pl.ds(g * LANES, LANES)] = jnp.float32(GAMMA) - acc

    pltpu.sync_copy(out_v, out_hbm.at[pl.ds(base, bpw)])


def _sc_score(ent_emb, cos_t, sin_t, idx_h, idx_r, idx_t):
    batch = idx_h.shape[0]
    assert batch % (8 * NW) == 0
    bpw = batch // NW
    dent = ent_emb.shape[1]
    mesh = plsc.VectorSubcoreMesh(core_axis_name="c", subcore_axis_name="s")
    kfn = functools.partial(
        pl.kernel,
        mesh=mesh,
        compiler_params=pltpu.CompilerParams(
            use_tc_tiling_on_sc=False, needs_layout_passes=False),
        out_type=jax.ShapeDtypeStruct((batch,), jnp.float32),
        scratch_types=[
            pltpu.VMEM((bpw,), jnp.int32),
            pltpu.VMEM((bpw,), jnp.int32),
            pltpu.VMEM((bpw,), jnp.int32),
            pltpu.VMEM((bpw, dent), jnp.float32),
            pltpu.VMEM((bpw, dent), jnp.float32),
            pltpu.VMEM((bpw, HID), jnp.float32),
            pltpu.VMEM((bpw, HID), jnp.float32),
            pltpu.VMEM((bpw,), jnp.float32),
            pltpu.SemaphoreType.DMA,
            pltpu.SemaphoreType.DMA,
            pltpu.SemaphoreType.DMA,
            pltpu.SemaphoreType.DMA,
        ],
    )(functools.partial(_sc_body, bpw))
    return kfn(ent_emb, cos_t, sin_t, idx_h, idx_r, idx_t)


def kernel(sample, ent_emb, rel_emb):
    cos_t, sin_t = _trig_tables(rel_emb)
    idx = sample.astype(jnp.int32)
    score = _sc_score(ent_emb, cos_t, sin_t, idx[:, 0], idx[:, 1], idx[:, 2])
    return score[:, None]


# speedup vs baseline: 1.9322x; 1.9322x over previous
import functools

import jax
import jax.numpy as jnp
from jax import lax
from jax.experimental import pallas as pl
from jax.experimental.pallas import tpu as pltpu
from jax.experimental.pallas import tpu_sc as plsc

HID = 128
GAMMA = 12.0
ERANGE = (12.0 + 2.0) / HID
PI = 3.141592653589793
PHASE_SCALE = PI / ERANGE

NC = 2
NS = 16
NW = NC * NS
LANES = 16


def _trig_body(rel_ref, cos_ref, sin_ref):
    phase = rel_ref[...] * jnp.float32(PHASE_SCALE)
    cos_ref[...] = jnp.cos(phase)
    sin_ref[...] = jnp.sin(phase)


def _trig_tables(rel_emb):
    n, h = rel_emb.shape
    out = jax.ShapeDtypeStruct((n, h), jnp.float32)
    return pl.pallas_call(_trig_body, out_shape=(out, out))(rel_emb)


def _sc_body(bpw, ent_hbm, cos_hbm, sin_hbm, idxh_hbm, idxr_hbm, idxt_hbm,
             out_hbm, idxh_v, idxr_v, idxt_v, h_v, t_v, c_v, s_v, out_v,
             sem0, sem1, sem2, sem3):
    wid = lax.axis_index("s") * NC + lax.axis_index("c")
    base = wid * bpw
    pltpu.sync_copy(idxh_hbm.at[pl.ds(base, bpw)], idxh_v)
    pltpu.sync_copy(idxr_hbm.at[pl.ds(base, bpw)], idxr_v)
    pltpu.sync_copy(idxt_hbm.at[pl.ds(base, bpw)], idxt_v)
    ch = pltpu.async_copy(ent_hbm.at[idxh_v], h_v, sem0)
    ct = pltpu.async_copy(ent_hbm.at[idxt_v], t_v, sem1)
    cc = pltpu.async_copy(cos_hbm.at[idxr_v], c_v, sem2)
    cs = pltpu.async_copy(sin_hbm.at[idxr_v], s_v, sem3)
    ch.wait()
    ct.wait()
    cc.wait()
    cs.wait()

    half = jnp.int32(HID)
    for g in range(bpw // LANES):
        row = lax.broadcasted_iota(jnp.int32, (LANES,), 0) + jnp.int32(g * LANES)

        def dim_step(d, acc, row=row):
            cd = jnp.full((LANES,), d, jnp.int32)
            cd2 = cd + half
            rh = plsc.load_gather(h_v, [row, cd])
            ih = plsc.load_gather(h_v, [row, cd2])
            rt = plsc.load_gather(t_v, [row, cd])
            it = plsc.load_gather(t_v, [row, cd2])
            c = plsc.load_gather(c_v, [row, cd])
            s = plsc.load_gather(s_v, [row, cd])
            re = rh * c - ih * s - rt
            im = rh * s + ih * c - it
            x = re * re + im * im + jnp.float32(1e-30)
            yi = jnp.int32(0x5F3759DF) - lax.shift_right_logical(
                plsc.bitcast(x, jnp.int32), jnp.int32(1))
            y = plsc.bitcast(yi, jnp.float32)
            hx = jnp.float32(0.5) * x
            y = y * (jnp.float32(1.5) - hx * y * y)
            y = y * (jnp.float32(1.5) - hx * y * y)
            y = y * (jnp.float32(1.5) - hx * y * y)
            return acc + x * y

        acc = lax.fori_loop(0, HID, dim_step, jnp.zeros((LANES,), jnp.float32))
        out_v[pl.ds(g * LANES, LANES)] = jnp.float32(GAMMA) - acc

    pltpu.sync_copy(out_v, out_hbm.at[pl.ds(base, bpw)])


def _sc_score(ent_emb, cos_t, sin_t, idx_h, idx_r, idx_t):
    batch = idx_h.shape[0]
    assert batch % (8 * NW) == 0
    bpw = batch // NW
    dent = ent_emb.shape[1]
    mesh = plsc.VectorSubcoreMesh(core_axis_name="c", subcore_axis_name="s")
    kfn = functools.partial(
        pl.kernel,
        mesh=mesh,
        compiler_params=pltpu.CompilerParams(needs_layout_passes=False),
        out_type=jax.ShapeDtypeStruct((batch,), jnp.float32),
        scratch_types=[
            pltpu.VMEM((bpw,), jnp.int32),
            pltpu.VMEM((bpw,), jnp.int32),
            pltpu.VMEM((bpw,), jnp.int32),
            pltpu.VMEM((bpw, dent), jnp.float32),
            pltpu.VMEM((bpw, dent), jnp.float32),
            pltpu.VMEM((bpw, HID), jnp.float32),
            pltpu.VMEM((bpw, HID), jnp.float32),
            pltpu.VMEM((bpw,), jnp.float32),
            pltpu.SemaphoreType.DMA,
            pltpu.SemaphoreType.DMA,
            pltpu.SemaphoreType.DMA,
            pltpu.SemaphoreType.DMA,
        ],
    )(functools.partial(_sc_body, bpw))
    return kfn(ent_emb, cos_t, sin_t, idx_h, idx_r, idx_t)


def kernel(sample, ent_emb, rel_emb):
    cos_t, sin_t = _trig_tables(rel_emb)
    idx = sample.astype(jnp.int32)
    score = _sc_score(ent_emb, cos_t, sin_t, idx[:, 0], idx[:, 1], idx[:, 2])
    return score[:, None]


# speedup vs baseline: 4.0390x; 2.0903x over previous
import functools

import jax
import jax.numpy as jnp
from jax import lax
from jax.experimental import pallas as pl
from jax.experimental.pallas import tpu as pltpu
from jax.experimental.pallas import tpu_sc as plsc

HID = 128
GAMMA = 12.0
ERANGE = (12.0 + 2.0) / HID
PI = 3.141592653589793
PHASE_SCALE = PI / ERANGE

NC = 2
NS = 16
NW = NC * NS
LANES = 16


def _trig_body(rel_ref, cos_ref, sin_ref):
    phase = rel_ref[...] * jnp.float32(PHASE_SCALE)
    cos_ref[...] = jnp.cos(phase)
    sin_ref[...] = jnp.sin(phase)


def _trig_tables(rel_emb):
    n, h = rel_emb.shape
    out = jax.ShapeDtypeStruct((n, h), jnp.float32)
    return pl.pallas_call(_trig_body, out_shape=(out, out))(rel_emb)


def _sc_body(bpw, ent_hbm, cos_hbm, sin_hbm, idxh_hbm, idxr_hbm, idxt_hbm,
             out_hbm, idxh_v, idxr_v, idxt_v, h_v, t_v, c_v, s_v, out_v,
             sem0, sem1, sem2, sem3):
    wid = lax.axis_index("s") * NC + lax.axis_index("c")
    base = wid * bpw
    pltpu.sync_copy(idxh_hbm.at[pl.ds(base, bpw)], idxh_v)
    pltpu.sync_copy(idxr_hbm.at[pl.ds(base, bpw)], idxr_v)
    pltpu.sync_copy(idxt_hbm.at[pl.ds(base, bpw)], idxt_v)
    ch = pltpu.async_copy(ent_hbm.at[idxh_v], h_v, sem0)
    ct = pltpu.async_copy(ent_hbm.at[idxt_v], t_v, sem1)
    cc = pltpu.async_copy(cos_hbm.at[idxr_v], c_v, sem2)
    cs = pltpu.async_copy(sin_hbm.at[idxr_v], s_v, sem3)
    ch.wait()
    ct.wait()
    cc.wait()
    cs.wait()

    lane = lax.broadcasted_iota(jnp.int32, (LANES,), 0)

    for g in range(bpw // LANES):
        def sample_step(k, vec, g=g):
            i = jnp.int32(g * LANES) + k
            chunks = []
            for j in range(HID // LANES):
                sl = pl.ds(j * LANES, LANES)
                sl2 = pl.ds(HID + j * LANES, LANES)
                rh = h_v[i, sl]
                ih = h_v[i, sl2]
                rt = t_v[i, sl]
                it = t_v[i, sl2]
                c = c_v[i, sl]
                s = s_v[i, sl]
                re = rh * c - ih * s - rt
                im = rh * s + ih * c - it
                x = re * re + im * im + jnp.float32(1e-30)
                yi = jnp.int32(0x5F3759DF) - lax.shift_right_logical(
                    plsc.bitcast(x, jnp.int32), jnp.int32(1))
                y = plsc.bitcast(yi, jnp.float32)
                hx = jnp.float32(0.5) * x
                y = y * (jnp.float32(1.5) - hx * y * y)
                y = y * (jnp.float32(1.5) - hx * y * y)
                y = y * (jnp.float32(1.5) - hx * y * y)
                chunks.append(x * y)
            while len(chunks) > 1:
                chunks = [a + b for a, b in zip(chunks[::2], chunks[1::2])]
            total = jnp.float32(GAMMA) - jnp.sum(chunks[0])
            return jnp.where(lane == k, total, vec)

        vec = lax.fori_loop(0, LANES, sample_step,
                            jnp.zeros((LANES,), jnp.float32))
        out_v[pl.ds(g * LANES, LANES)] = vec

    pltpu.sync_copy(out_v, out_hbm.at[pl.ds(base, bpw)])


def _sc_score(ent_emb, cos_t, sin_t, idx_h, idx_r, idx_t):
    batch = idx_h.shape[0]
    assert batch % (8 * NW) == 0
    bpw = batch // NW
    dent = ent_emb.shape[1]
    mesh = plsc.VectorSubcoreMesh(core_axis_name="c", subcore_axis_name="s")
    kfn = functools.partial(
        pl.kernel,
        mesh=mesh,
        compiler_params=pltpu.CompilerParams(needs_layout_passes=False),
        out_type=jax.ShapeDtypeStruct((batch,), jnp.float32),
        scratch_types=[
            pltpu.VMEM((bpw,), jnp.int32),
            pltpu.VMEM((bpw,), jnp.int32),
            pltpu.VMEM((bpw,), jnp.int32),
            pltpu.VMEM((bpw, dent), jnp.float32),
            pltpu.VMEM((bpw, dent), jnp.float32),
            pltpu.VMEM((bpw, HID), jnp.float32),
            pltpu.VMEM((bpw, HID), jnp.float32),
            pltpu.VMEM((bpw,), jnp.float32),
            pltpu.SemaphoreType.DMA,
            pltpu.SemaphoreType.DMA,
            pltpu.SemaphoreType.DMA,
            pltpu.SemaphoreType.DMA,
        ],
    )(functools.partial(_sc_body, bpw))
    return kfn(ent_emb, cos_t, sin_t, idx_h, idx_r, idx_t)


def kernel(sample, ent_emb, rel_emb):
    cos_t, sin_t = _trig_tables(rel_emb)
    idx = sample.astype(jnp.int32)
    score = _sc_score(ent_emb, cos_t, sin_t, idx[:, 0], idx[:, 1], idx[:, 2])
    return score[:, None]


# speedup vs baseline: 4.5998x; 1.1389x over previous
import functools

import jax
import jax.numpy as jnp
from jax import lax
from jax.experimental import pallas as pl
from jax.experimental.pallas import tpu as pltpu
from jax.experimental.pallas import tpu_sc as plsc

HID = 128
GAMMA = 12.0
ERANGE = (12.0 + 2.0) / HID
PI = 3.141592653589793
PHASE_SCALE = PI / ERANGE

NC = 2
NS = 16
NW = NC * NS
LANES = 16
NBLK = 4


def _trig_body(rel_ref, trig_ref):
    phase = rel_ref[...] * jnp.float32(PHASE_SCALE)
    trig_ref[:, :HID] = jnp.cos(phase)
    trig_ref[:, HID:] = jnp.sin(phase)


def _trig_table(rel_emb):
    n, h = rel_emb.shape
    return pl.pallas_call(
        _trig_body,
        out_shape=jax.ShapeDtypeStruct((n, 2 * h), jnp.float32),
    )(rel_emb)


def _score_block(h_v, t_v, c_v, out_v, blk, bs):
    lane = lax.broadcasted_iota(jnp.int32, (LANES,), 0)
    for g in range(bs // LANES):
        def sample_step(k, vec, g=g):
            i = jnp.int32(g * LANES) + k
            chunks = []
            for j in range(HID // LANES):
                sl = pl.ds(j * LANES, LANES)
                sl2 = pl.ds(HID + j * LANES, LANES)
                rh = h_v[i, sl]
                ih = h_v[i, sl2]
                rt = t_v[i, sl]
                it = t_v[i, sl2]
                c = c_v[i, sl]
                s = c_v[i, sl2]
                re = rh * c - ih * s - rt
                im = rh * s + ih * c - it
                x = re * re + im * im + jnp.float32(1e-30)
                yi = jnp.int32(0x5F3759DF) - lax.shift_right_logical(
                    plsc.bitcast(x, jnp.int32), jnp.int32(1))
                y = plsc.bitcast(yi, jnp.float32)
                hx = jnp.float32(0.5) * x
                y = y * (jnp.float32(1.5) - hx * y * y)
                y = y * (jnp.float32(1.5) - hx * y * y)
                chunks.append(x * y)
            while len(chunks) > 1:
                chunks = [a + b for a, b in zip(chunks[::2], chunks[1::2])]
            total = jnp.float32(GAMMA) - jnp.sum(chunks[0])
            return jnp.where(lane == k, total, vec)

        vec = lax.fori_loop(0, LANES, sample_step,
                            jnp.zeros((LANES,), jnp.float32))
        out_v[pl.ds(blk * bs + g * LANES, LANES)] = vec


def _sc_body(bpw, ent_hbm, trig_hbm, idxh_hbm, idxr_hbm, idxt_hbm,
             out_hbm, idxh_v, idxr_v, idxt_v,
             h0_v, h1_v, t0_v, t1_v, c0_v, c1_v, out_v,
             sem0, sem1, semi):
    bs = bpw // NBLK
    wid = lax.axis_index("s") * NC + lax.axis_index("c")
    base = wid * bpw
    ci = pltpu.async_copy(idxh_hbm.at[pl.ds(base, bpw)], idxh_v, semi)
    cr = pltpu.async_copy(idxr_hbm.at[pl.ds(base, bpw)], idxr_v, semi)
    ct = pltpu.async_copy(idxt_hbm.at[pl.ds(base, bpw)], idxt_v, semi)
    ci.wait()
    cr.wait()
    ct.wait()

    hb = [h0_v, h1_v]
    tb = [t0_v, t1_v]
    cb = [c0_v, c1_v]
    sems = [sem0, sem1]

    def fire(blk):
        b = blk % 2
        s = pl.ds(blk * bs, bs)
        return (
            pltpu.async_copy(ent_hbm.at[idxh_v.at[s]], hb[b], sems[b]),
            pltpu.async_copy(ent_hbm.at[idxt_v.at[s]], tb[b], sems[b]),
            pltpu.async_copy(trig_hbm.at[idxr_v.at[s]], cb[b], sems[b]),
        )

    inflight = fire(0)
    for blk in range(NBLK):
        cur = inflight
        if blk + 1 < NBLK:
            inflight = fire(blk + 1)
        for c in cur:
            c.wait()
        b = blk % 2
        _score_block(hb[b], tb[b], cb[b], out_v, blk, bs)

    pltpu.sync_copy(out_v, out_hbm.at[pl.ds(base, bpw)])


def _sc_score(ent_emb, trig_t, idx_h, idx_r, idx_t):
    batch = idx_h.shape[0]
    assert batch % (8 * NW) == 0
    bpw = batch // NW
    bs = bpw // NBLK
    dent = ent_emb.shape[1]
    mesh = plsc.VectorSubcoreMesh(core_axis_name="c", subcore_axis_name="s")
    kfn = functools.partial(
        pl.kernel,
        mesh=mesh,
        compiler_params=pltpu.CompilerParams(needs_layout_passes=False),
        out_type=jax.ShapeDtypeStruct((batch,), jnp.float32),
        scratch_types=[
            pltpu.VMEM((bpw,), jnp.int32),
            pltpu.VMEM((bpw,), jnp.int32),
            pltpu.VMEM((bpw,), jnp.int32),
            pltpu.VMEM((bs, dent), jnp.float32),
            pltpu.VMEM((bs, dent), jnp.float32),
            pltpu.VMEM((bs, dent), jnp.float32),
            pltpu.VMEM((bs, dent), jnp.float32),
            pltpu.VMEM((bs, 2 * HID), jnp.float32),
            pltpu.VMEM((bs, 2 * HID), jnp.float32),
            pltpu.VMEM((bpw,), jnp.float32),
            pltpu.SemaphoreType.DMA,
            pltpu.SemaphoreType.DMA,
            pltpu.SemaphoreType.DMA,
        ],
    )(functools.partial(_sc_body, bpw))
    return kfn(ent_emb, trig_t, idx_h, idx_r, idx_t)


def kernel(sample, ent_emb, rel_emb):
    trig_t = _trig_table(rel_emb)
    idx = sample.astype(jnp.int32)
    score = _sc_score(ent_emb, trig_t, idx[:, 0], idx[:, 1], idx[:, 2])
    return score[:, None]
